# baseline (speedup 1.0000x reference)
import functools

import jax
import jax.numpy as jnp
from jax import lax
from jax.experimental import pallas as pl
from jax.experimental.pallas import tpu as pltpu
from jax.experimental.pallas import tpu_sc as plsc

NUM_TOKENS = 1000000
EMBED_DIM = 64
BATCH = 4096
SEQ = 200

NC = 2
NS = 16
NW = NC * NS

B_TOTAL = BATCH * SEQ
PER_W = B_TOTAL // NW
CHUNK = 128
N_STEPS = PER_W // CHUNK
N_PAIRS = N_STEPS // 2


@jax.jit
def _embed_gather(token_ids, emb_table):
    mesh = plsc.VectorSubcoreMesh(core_axis_name="c", subcore_axis_name="s")

    flat = token_ids.reshape(-1)
    pidx3 = jax.lax.shift_right_logical(flat, 1).reshape(NW, N_STEPS, CHUNK)
    h3 = jax.lax.shift_left(jnp.bitwise_and(flat, 1), 6).reshape(
        NW, N_STEPS, CHUNK
    )
    tbl2 = jnp.concatenate([emb_table[0::2], emb_table[1::2]], axis=1)

    @functools.partial(
        pl.kernel,
        mesh=mesh,
        out_type=jax.ShapeDtypeStruct((B_TOTAL, EMBED_DIM), jnp.float32),
        scratch_types=[
            pltpu.VMEM((N_STEPS, CHUNK), jnp.int32),
            pltpu.VMEM((N_STEPS, CHUNK), jnp.int32),
            pltpu.VMEM((CHUNK, 2 * EMBED_DIM), jnp.float32),
            pltpu.VMEM((CHUNK, 2 * EMBED_DIM), jnp.float32),
            pltpu.VMEM((CHUNK, EMBED_DIM), jnp.float32),
            pltpu.VMEM((CHUNK, EMBED_DIM), jnp.float32),
            pltpu.SemaphoreType.DMA,
            pltpu.SemaphoreType.DMA,
            pltpu.SemaphoreType.DMA,
            pltpu.SemaphoreType.DMA,
        ],
    )
    def k(pidx_hbm, h_hbm, tbl_hbm, out_hbm,
          pidx_v, h_v, buf0, buf1, rb0, rb1, g0, g1, w0, w1):
        wid = lax.axis_index("s") * NC + lax.axis_index("c")
        base = wid * PER_W
        pltpu.sync_copy(pidx_hbm.at[wid], pidx_v)
        pltpu.sync_copy(h_hbm.at[wid], h_v)

        def gather(j, buf, sem):
            pltpu.async_copy(tbl_hbm.at[pidx_v.at[j]], buf, sem)

        def gather_wait(j, buf, sem):
            pltpu.make_async_copy(tbl_hbm.at[pidx_v.at[j]], buf, sem).wait()

        def write(j, rb, sem):
            pltpu.async_copy(
                rb, out_hbm.at[pl.ds(base + j * CHUNK, CHUNK)], sem
            )

        def write_wait(j, rb, sem):
            pltpu.make_async_copy(
                rb, out_hbm.at[pl.ds(base + j * CHUNK, CHUNK)], sem
            ).wait()

        def select(j, buf, rb):
            for t0 in range(0, CHUNK, 16):
                hv = h_v[j, pl.ds(t0, 16)]
                for ti in range(16):
                    t = t0 + ti
                    h = hv[ti]
                    for m in range(EMBED_DIM // 16):
                        rb[t, pl.ds(16 * m, 16)] = buf[t, pl.ds(h + 16 * m, 16)]

        gather(0, buf0, g0)

        def body(p, _):
            j0 = 2 * p
            gather_wait(j0, buf0, g0)
            gather(j0 + 1, buf1, g1)
            pl.when(p > 0)(lambda: write_wait(j0 - 2, rb0, w0))
            select(j0, buf0, rb0)
            write(j0, rb0, w0)

            gather_wait(j0 + 1, buf1, g1)
            pl.when(p + 1 < N_PAIRS)(lambda: gather(j0 + 2, buf0, g0))
            pl.when(p > 0)(lambda: write_wait(j0 - 1, rb1, w1))
            select(j0 + 1, buf1, rb1)
            write(j0 + 1, rb1, w1)
            return 0

        lax.fori_loop(0, N_PAIRS, body, 0)
        write_wait(N_STEPS - 2, rb0, w0)
        write_wait(N_STEPS - 1, rb1, w1)

    q = k(pidx3, h3, tbl2)
    return q.reshape(BATCH, SEQ, EMBED_DIM)


def kernel(token_ids, emb_table):
    return _embed_gather(token_ids, emb_table)


# speedup vs baseline: 8.0274x; 8.0274x over previous
import functools

import jax
import jax.numpy as jnp
from jax import lax
from jax.experimental import pallas as pl
from jax.experimental.pallas import tpu as pltpu
from jax.experimental.pallas import tpu_sc as plsc

NUM_TOKENS = 1000000
EMBED_DIM = 64
BATCH = 4096
SEQ = 200

NC = 2
NS = 16
NW = NC * NS

B_TOTAL = BATCH * SEQ
PER_W = B_TOTAL // NW
CHUNK = 128
N_STEPS = PER_W // CHUNK
N_PAIRS = N_STEPS // 2


@jax.jit
def _embed_gather(token_ids, emb_table):
    mesh = plsc.VectorSubcoreMesh(core_axis_name="c", subcore_axis_name="s")

    flat = token_ids.reshape(-1)
    pidx3 = jax.lax.shift_right_logical(flat, 1).reshape(NW, N_STEPS, CHUNK)
    h3 = jnp.bitwise_and(flat, 1).astype(jnp.float32).reshape(
        NW, N_STEPS, CHUNK
    )
    tbl2 = emb_table.reshape(NUM_TOKENS // 2, 2 * EMBED_DIM)

    @functools.partial(
        pl.kernel,
        mesh=mesh,
        out_type=jax.ShapeDtypeStruct((B_TOTAL, EMBED_DIM), jnp.float32),
        scratch_types=[
            pltpu.VMEM((N_STEPS, CHUNK), jnp.int32),
            pltpu.VMEM((N_STEPS, CHUNK), jnp.float32),
            pltpu.VMEM((CHUNK, 2 * EMBED_DIM), jnp.float32),
            pltpu.VMEM((CHUNK, 2 * EMBED_DIM), jnp.float32),
            pltpu.VMEM((CHUNK, EMBED_DIM), jnp.float32),
            pltpu.VMEM((CHUNK, EMBED_DIM), jnp.float32),
            pltpu.SemaphoreType.DMA,
            pltpu.SemaphoreType.DMA,
            pltpu.SemaphoreType.DMA,
            pltpu.SemaphoreType.DMA,
        ],
    )
    def k(pidx_hbm, h_hbm, tbl_hbm, out_hbm,
          pidx_v, h_v, buf0, buf1, rb0, rb1, g0, g1, w0, w1):
        wid = lax.axis_index("s") * NC + lax.axis_index("c")
        base = wid * PER_W
        pltpu.sync_copy(pidx_hbm.at[wid], pidx_v)
        pltpu.sync_copy(h_hbm.at[wid], h_v)

        def gather(j, buf, sem):
            pltpu.async_copy(tbl_hbm.at[pidx_v.at[j]], buf, sem)

        def gather_wait(j, buf, sem):
            pltpu.make_async_copy(tbl_hbm.at[pidx_v.at[j]], buf, sem).wait()

        def write(j, rb, sem):
            pltpu.async_copy(
                rb, out_hbm.at[pl.ds(base + j * CHUNK, CHUNK)], sem
            )

        def write_wait(j, rb, sem):
            pltpu.make_async_copy(
                rb, out_hbm.at[pl.ds(base + j * CHUNK, CHUNK)], sem
            ).wait()

        def select(j, buf, rb):
            for t0 in range(0, CHUNK, 16):
                hv = h_v[j, pl.ds(t0, 16)]
                for ti in range(16):
                    t = t0 + ti
                    sp = jnp.full((16,), ti, jnp.int32)
                    w = hv.at[sp].get(mode="promise_in_bounds")
                    wn = 1.0 - w
                    for seg in range(EMBED_DIM // 16):
                        lo = buf[t, pl.ds(16 * seg, 16)]
                        hi = buf[t, pl.ds(EMBED_DIM + 16 * seg, 16)]
                        rb[t, pl.ds(16 * seg, 16)] = lo * wn + hi * w

        gather(0, buf0, g0)

        def body(p, _):
            j0 = 2 * p
            gather_wait(j0, buf0, g0)
            gather(j0 + 1, buf1, g1)
            pl.when(p > 0)(lambda: write_wait(j0 - 2, rb0, w0))
            select(j0, buf0, rb0)
            write(j0, rb0, w0)

            gather_wait(j0 + 1, buf1, g1)
            pl.when(p + 1 < N_PAIRS)(lambda: gather(j0 + 2, buf0, g0))
            pl.when(p > 0)(lambda: write_wait(j0 - 1, rb1, w1))
            select(j0 + 1, buf1, rb1)
            write(j0 + 1, rb1, w1)
            return 0

        lax.fori_loop(0, N_PAIRS, body, 0)
        write_wait(N_STEPS - 2, rb0, w0)
        write_wait(N_STEPS - 1, rb1, w1)

    q = k(pidx3, h3, tbl2)
    return q.reshape(BATCH, SEQ, EMBED_DIM)


def kernel(token_ids, emb_table):
    return _embed_gather(token_ids, emb_table)
